# baseline (speedup 1.0000x reference)
import functools

import jax
import jax.numpy as jnp
from jax import lax
from jax.experimental import pallas as pl
from jax.experimental.pallas import tpu as pltpu
from jax.experimental.pallas import tpu_sc as plsc

NCORE = 2
NSUB = 16
NW = NCORE * NSUB
LANES = 16
D = 64
CHUNK = 128
KNEG = 8
MARGIN = 1.0


def _tec_body(h_hbm, t_hbm, r_hbm, ent_hbm, rel_hbm, out_hbm,
              idx_h, idx_t, idx_r, rows_h, rows_t, rows_r,
              scores_p, scores_n, loss_buf, sem):
    cid = lax.axis_index("c")
    sid = lax.axis_index("s")
    wid = sid * NCORE + cid
    lane = lax.iota(jnp.int32, LANES)

    def gather_rows(base):
        pltpu.sync_copy(h_hbm.at[pl.ds(base, CHUNK)], idx_h)
        pltpu.sync_copy(t_hbm.at[pl.ds(base, CHUNK)], idx_t)
        pltpu.sync_copy(r_hbm.at[pl.ds(base, CHUNK)], idx_r)
        pltpu.async_copy(ent_hbm.at[idx_h], rows_h, sem).wait()
        pltpu.async_copy(ent_hbm.at[idx_t], rows_t, sem).wait()
        pltpu.async_copy(rel_hbm.at[idx_r], rows_r, sem).wait()

    def score_chunk(scores_ref):
        def g_body(g, carry):
            row0 = g * LANES + lane
            flat0 = row0 * D

            def d_body(dd, acc):
                col = jnp.full((LANES,), dd, jnp.int32)
                hv = plsc.load_gather(rows_h, [row0, col])
                rv = plsc.load_gather(rows_r, [row0, col])
                tv = plsc.load_gather(rows_t, [row0, col])
                return acc + jnp.abs(hv + rv - tv)

            acc = lax.fori_loop(0, D, d_body, jnp.zeros((LANES,), jnp.float32),
                                unroll=8)
            plsc.store_scatter(scores_ref, [row0], acc)
            return carry

        lax.fori_loop(0, CHUNK // LANES, g_body, jnp.int32(0))

    gather_rows(wid * CHUNK)
    score_chunk(scores_p)

    npos = 4096
    loss_acc = jnp.zeros((LANES,), jnp.float32)
    for j in range(KNEG):
        gather_rows(npos + wid * (CHUNK * KNEG) + j * CHUNK)
        score_chunk(scores_n)
        nacc = jnp.zeros((LANES,), jnp.float32)
        for k in range(KNEG):
            nacc = nacc + plsc.load_gather(scores_n, [lane * KNEG + k])
        p = scores_p[pl.ds(j * LANES, LANES)]
        loss_acc = loss_acc + jnp.maximum(p - nacc * (1.0 / KNEG) + MARGIN, 0.0)

    loss_buf[...] = loss_acc
    pltpu.sync_copy(loss_buf, out_hbm.at[wid])


@functools.partial(jax.jit, static_argnums=())
def kernel(batch_h, batch_t, batch_r, batch_size, n_negative,
           ent_embeddings, rel_embeddings):
    del batch_size, n_negative
    mesh = plsc.VectorSubcoreMesh(core_axis_name="c", subcore_axis_name="s",
                                  num_cores=NCORE, num_subcores=NSUB)
    kern = pl.kernel(
        _tec_body,
        out_type=jax.ShapeDtypeStruct((NW, LANES), jnp.float32),
        mesh=mesh,
        compiler_params=pltpu.CompilerParams(needs_layout_passes=False,
                                             use_tc_tiling_on_sc=False),
        scratch_types=[
            pltpu.VMEM((CHUNK,), jnp.int32),
            pltpu.VMEM((CHUNK,), jnp.int32),
            pltpu.VMEM((CHUNK,), jnp.int32),
            pltpu.VMEM((CHUNK, D), jnp.float32),
            pltpu.VMEM((CHUNK, D), jnp.float32),
            pltpu.VMEM((CHUNK, D), jnp.float32),
            pltpu.VMEM((CHUNK,), jnp.float32),
            pltpu.VMEM((CHUNK,), jnp.float32),
            pltpu.VMEM((LANES,), jnp.float32),
            pltpu.SemaphoreType.DMA,
        ],
    )
    partials = kern(batch_h, batch_t, batch_r, ent_embeddings, rel_embeddings)
    return jnp.sum(partials)
